# Initial kernel scaffold; baseline (speedup 1.0000x reference)
import jax
import jax.numpy as jnp
from jax.experimental import pallas as pl


def kernel(ctrl_pts, Jm_array, tensor_prod, num_supp_bs_cumsum):
    raise NotImplementedError("write your pallas kernel here")


# speedup vs baseline: 1146.2833x; 1146.2833x over previous
import functools

import jax
import jax.numpy as jnp
from jax import lax
from jax.experimental import pallas as pl
from jax.experimental.pallas import tpu as pltpu
from jax.experimental.pallas import tpu_sc as plsc

_LANES = 16
_CHUNK = 8192
_IDXW = 128
_ROWS = _CHUNK // _IDXW
_ALIGN = 8 * _IDXW


def _sc_body(ns, total_supp, seg_w,
             ctrl8, jm2, tp, cpad, out,
             cseg, idx2d, tpv, rows, acc,
             sem_i, sem_t, sem_g):
  iota = lax.iota(jnp.int32, _LANES)
  col1 = jnp.full((_LANES,), 1, jnp.int32)
  col2 = jnp.full((_LANES,), 2, jnp.int32)
  col0 = jnp.zeros((_LANES,), jnp.int32)
  m3 = iota < 3
  fzero = jnp.zeros((_LANES,), jnp.float32)
  max_row0 = total_supp // _IDXW - _ROWS

  wid = lax.axis_index("c") * ns + lax.axis_index("s")
  s0 = wid * seg_w

  pltpu.sync_copy(cpad.at[pl.ds(s0, seg_w + 32)], cseg)

  def _cs(i):
    return cseg[pl.ds(i, _LANES)][0]

  p1 = _cs(seg_w)

  def seg_body(s_, wbase):
    cs = _cs(s_)
    cn = _cs(s_ + 1)
    plen = cn - cs
    nsteps = (plen + (_LANES - 1)) // _LANES

    def step(i, carry):
      wb, vx, vy, vz = carry
      bs = cs + i * _LANES
      need_end = jnp.minimum(bs + _LANES, p1)
      refill = need_end > wb + _CHUNK
      new_base = bs & ~jnp.int32(_ALIGN - 1)
      row0 = jnp.minimum(new_base // _IDXW, max_row0)
      row0 = pl.multiple_of(row0, 8)
      wb_new = jnp.where(refill, row0 * _IDXW, wb)

      @pl.when(refill)
      def _():
        cp_i = pltpu.async_copy(jm2.at[pl.ds(row0, _ROWS)], idx2d, sem_i)
        cp_t = pltpu.async_copy(tp.at[pl.ds(row0 * _IDXW, _CHUNK)], tpv,
                                sem_t)
        cp_i.wait()

        def fire(j, _):
          pltpu.async_copy(ctrl8.at[idx2d.at[j]],
                           rows.at[pl.ds(j * _IDXW, _IDXW)], sem_g)
          return 0

        lax.fori_loop(0, _ROWS, fire, 0)
        cp_t.wait()
        pltpu.make_async_copy(ctrl8.at[idx2d.at[0]], rows, sem_g).wait()

      off = (bs - wb_new) + iota
      valid = (i * _LANES + iota) < plen
      tpg = plsc.load_gather(tpv, [off], mask=valid)
      tpm = jnp.where(valid, tpg, 0.0)
      gx = plsc.load_gather(rows, [off, col0], mask=valid)
      gy = plsc.load_gather(rows, [off, col1], mask=valid)
      gz = plsc.load_gather(rows, [off, col2], mask=valid)
      vx = vx + tpm * jnp.where(valid, gx, 0.0)
      vy = vy + tpm * jnp.where(valid, gy, 0.0)
      vz = vz + tpm * jnp.where(valid, gz, 0.0)
      return (wb_new, vx, vy, vz)

    wbase, vx, vy, vz = lax.fori_loop(0, nsteps, step,
                                      (wbase, fzero, fzero, fzero))
    sx = jnp.sum(vx)
    sy = jnp.sum(vy)
    sz = jnp.sum(vz)
    contrib = (jnp.where(iota == 0, sx, 0.0)
               + jnp.where(iota == 1, sy, 0.0)
               + jnp.where(iota == 2, sz, 0.0))
    plsc.store_scatter(acc, [4 * s_ + iota], contrib, mask=m3)
    return wbase

  lax.fori_loop(0, seg_w, seg_body, jnp.int32(-(2 ** 30)))

  pltpu.sync_copy(acc.at[pl.ds(0, seg_w * 4)],
                  out.at[pl.ds(wid * seg_w * 4, seg_w * 4)])


def kernel(ctrl_pts, Jm_array, tensor_prod, num_supp_bs_cumsum):
  total_supp = Jm_array.shape[0]
  num_eval = num_supp_bs_cumsum.shape[0] - 1

  try:
    info = plsc.get_sparse_core_info()
    nc, ns = info.num_cores, info.num_subcores
  except ValueError:
    nc, ns = 2, 16
  nw = nc * ns
  seg_w = num_eval // nw
  assert num_eval % nw == 0 and total_supp % _CHUNK == 0

  ctrl8 = jnp.pad(ctrl_pts, ((0, 0), (0, 5)))
  jm2 = Jm_array.reshape(total_supp // _IDXW, _IDXW)
  cpad = jnp.pad(num_supp_bs_cumsum, (0, 32))

  accw = seg_w * 4 + _LANES

  mesh = plsc.VectorSubcoreMesh(core_axis_name="c", subcore_axis_name="s",
                                num_cores=nc, num_subcores=ns)
  out_flat = pl.kernel(
      functools.partial(_sc_body, ns, total_supp, seg_w),
      out_type=jax.ShapeDtypeStruct((num_eval * 4,), jnp.float32),
      mesh=mesh,
      compiler_params=pltpu.CompilerParams(needs_layout_passes=False,
                                           use_tc_tiling_on_sc=False),
      scratch_types=[
          pltpu.VMEM((seg_w + 32,), jnp.int32),
          pltpu.VMEM((_ROWS, _IDXW), jnp.int32),
          pltpu.VMEM((_CHUNK,), jnp.float32),
          pltpu.VMEM((_CHUNK, 8), jnp.float32),
          pltpu.VMEM((accw,), jnp.float32),
          pltpu.SemaphoreType.DMA,
          pltpu.SemaphoreType.DMA,
          pltpu.SemaphoreType.DMA,
      ],
  )(ctrl8, jm2, tensor_prod, cpad)

  return out_flat.reshape(num_eval, 4)[:, :3]


# speedup vs baseline: 1422.4566x; 1.2409x over previous
import functools

import jax
import jax.numpy as jnp
from jax import lax
from jax.experimental import pallas as pl
from jax.experimental.pallas import tpu as pltpu
from jax.experimental.pallas import tpu_sc as plsc

_LANES = 16
_CHUNK = 4096
_IDXW = 128
_ROWS = _CHUNK // _IDXW
_ALIGN = 8 * _IDXW


def _sc_body(ns, total_supp, seg_w,
             ctrl8, jm2, tp, cpad, out,
             shtab, cseg, idx2d, tpv, rows, acc,
             sem_i, sem_t, sem_g):
  iota = lax.iota(jnp.int32, _LANES)
  col1 = jnp.full((_LANES,), 1, jnp.int32)
  col2 = jnp.full((_LANES,), 2, jnp.int32)
  col0 = jnp.zeros((_LANES,), jnp.int32)
  m3 = iota < 3
  fzero = jnp.zeros((_LANES,), jnp.float32)
  max_row0 = total_supp // _IDXW - _ROWS

  sid = lax.axis_index("s")
  wid = lax.axis_index("c") * ns + sid
  s0 = wid * seg_w

  @pl.when(sid == 0)
  def _():
    pltpu.sync_copy(ctrl8, shtab)

  plsc.subcore_barrier()

  pltpu.sync_copy(cpad.at[pl.ds(s0, seg_w + 32)], cseg)

  def _cs(i):
    return cseg[pl.ds(i, _LANES)][0]

  p1 = _cs(seg_w)

  def seg_body(s_, wbase):
    cs = _cs(s_)
    cn = _cs(s_ + 1)
    plen = cn - cs
    nsteps = (plen + (_LANES - 1)) // _LANES

    def step(i, carry):
      wb, vx, vy, vz = carry
      bs = cs + i * _LANES
      need_end = jnp.minimum(bs + _LANES, p1)
      refill = need_end > wb + _CHUNK
      new_base = bs & ~jnp.int32(_ALIGN - 1)
      row0 = jnp.minimum(new_base // _IDXW, max_row0)
      row0 = pl.multiple_of(row0, 8)
      wb_new = jnp.where(refill, row0 * _IDXW, wb)

      @pl.when(refill)
      def _():
        cp_i = pltpu.async_copy(jm2.at[pl.ds(row0, _ROWS)], idx2d, sem_i)
        cp_t = pltpu.async_copy(tp.at[pl.ds(row0 * _IDXW, _CHUNK)], tpv,
                                sem_t)
        cp_i.wait()

        def fire(j, _):
          pltpu.async_copy(shtab.at[idx2d.at[j]],
                           rows.at[pl.ds(j * _IDXW, _IDXW)], sem_g)
          return 0

        lax.fori_loop(0, _ROWS, fire, 0)
        cp_t.wait()
        pltpu.make_async_copy(ctrl8.at[idx2d.at[0]], rows, sem_g).wait()

      off = (bs - wb_new) + iota
      valid = (i * _LANES + iota) < plen
      tpg = plsc.load_gather(tpv, [off], mask=valid)
      tpm = jnp.where(valid, tpg, 0.0)
      gx = plsc.load_gather(rows, [off, col0], mask=valid)
      gy = plsc.load_gather(rows, [off, col1], mask=valid)
      gz = plsc.load_gather(rows, [off, col2], mask=valid)
      vx = vx + tpm * jnp.where(valid, gx, 0.0)
      vy = vy + tpm * jnp.where(valid, gy, 0.0)
      vz = vz + tpm * jnp.where(valid, gz, 0.0)
      return (wb_new, vx, vy, vz)

    wbase, vx, vy, vz = lax.fori_loop(0, nsteps, step,
                                      (wbase, fzero, fzero, fzero))
    sx = jnp.sum(vx)
    sy = jnp.sum(vy)
    sz = jnp.sum(vz)
    contrib = (jnp.where(iota == 0, sx, 0.0)
               + jnp.where(iota == 1, sy, 0.0)
               + jnp.where(iota == 2, sz, 0.0))
    plsc.store_scatter(acc, [4 * s_ + iota], contrib, mask=m3)
    return wbase

  lax.fori_loop(0, seg_w, seg_body, jnp.int32(-(2 ** 30)))

  pltpu.sync_copy(acc.at[pl.ds(0, seg_w * 4)],
                  out.at[pl.ds(wid * seg_w * 4, seg_w * 4)])


def kernel(ctrl_pts, Jm_array, tensor_prod, num_supp_bs_cumsum):
  total_supp = Jm_array.shape[0]
  num_eval = num_supp_bs_cumsum.shape[0] - 1

  try:
    info = plsc.get_sparse_core_info()
    nc, ns = info.num_cores, info.num_subcores
  except ValueError:
    nc, ns = 2, 16
  nw = nc * ns
  seg_w = num_eval // nw
  assert num_eval % nw == 0 and total_supp % _CHUNK == 0

  ctrl8 = jnp.pad(ctrl_pts, ((0, 0), (0, 5)))
  jm2 = Jm_array.reshape(total_supp // _IDXW, _IDXW)
  cpad = jnp.pad(num_supp_bs_cumsum, (0, 32))

  accw = seg_w * 4 + _LANES

  mesh = plsc.VectorSubcoreMesh(core_axis_name="c", subcore_axis_name="s",
                                num_cores=nc, num_subcores=ns)
  out_flat = pl.kernel(
      functools.partial(_sc_body, ns, total_supp, seg_w),
      out_type=jax.ShapeDtypeStruct((num_eval * 4,), jnp.float32),
      mesh=mesh,
      compiler_params=pltpu.CompilerParams(needs_layout_passes=False,
                                           use_tc_tiling_on_sc=False),
      scratch_types=[
          pltpu.VMEM_SHARED((ctrl8.shape[0], 8), jnp.float32),
          pltpu.VMEM((seg_w + 32,), jnp.int32),
          pltpu.VMEM((_ROWS, _IDXW), jnp.int32),
          pltpu.VMEM((_CHUNK,), jnp.float32),
          pltpu.VMEM((_CHUNK, 8), jnp.float32),
          pltpu.VMEM((accw,), jnp.float32),
          pltpu.SemaphoreType.DMA,
          pltpu.SemaphoreType.DMA,
          pltpu.SemaphoreType.DMA,
      ],
  )(ctrl8, jm2, tensor_prod, cpad)

  return out_flat.reshape(num_eval, 4)[:, :3]
